# baseline (speedup 1.0000x reference)
import jax
import jax.numpy as jnp
from jax import lax
from jax.experimental import pallas as pl
from jax.experimental.pallas import tpu as pltpu

NUM_FEATURES = 100
NUM_FIELDS = 26
EMBED = 16
EH = 8
CB = 1024


def _x_copy(xt_hbm, xbuf, sem, step):
    slot = lax.rem(step, 2)
    return pltpu.make_async_copy(
        xt_hbm.at[:, pl.ds(CB * step, CB)],
        xbuf.at[slot],
        sem.at[slot])


def _fm_body(xt_hbm, w_ref, V_ref, fi_ref, yfm_ref, out_ref,
             embs_ref, a_ref, q_ref, xbuf, sem):
    f32 = jnp.float32
    hi = lax.Precision.HIGHEST
    i = pl.program_id(0)
    j = pl.program_id(1)
    n = pl.num_programs(0)

    @pl.when((i == 0) & (j == 0))
    def _prep():
        _x_copy(xt_hbm, xbuf, sem, jnp.int32(0)).start()
        fi = fi_ref[:]
        onehot = (fi == lax.broadcasted_iota(
            jnp.int32, (NUM_FEATURES, NUM_FIELDS), 1)).astype(f32)
        emb = jnp.dot(onehot, V_ref[:], precision=hi,
                      preferred_element_type=f32)
        embs_ref[0] = emb[:, :EH]
        embs_ref[1] = emb[:, EH:]
        rowsum = jnp.sum(emb, axis=1, keepdims=True)
        a_ref[:] = jnp.concatenate([w_ref[:], rowsum], axis=1)
        q_ref[:] = jnp.sum(emb * emb, axis=1, keepdims=True)

    @pl.when(j == 0)
    def _wait_x():
        _x_copy(xt_hbm, xbuf, sem, i).wait()

    @pl.when((j == 1) & (i + 1 < n))
    def _prefetch():
        _x_copy(xt_hbm, xbuf, sem, i + 1).start()

    xtb = xbuf[lax.rem(i, 2)]
    emb_h = embs_ref[j]
    for e in range(EH):
        out_ref[:, e, :] = xtb * emb_h[:, e:e + 1]

    @pl.when(j == 0)
    def _yfm():
        p = lax.dot_general(a_ref[:], xtb, (((0,), (0,)), ((), ())),
                            precision=hi, preferred_element_type=f32)
        sq = lax.dot_general(q_ref[:], xtb * xtb, (((0,), (0,)), ((), ())),
                             precision=hi, preferred_element_type=f32)
        inter = 0.5 * (p[1:2] * p[1:2] - sq)
        yfm_ref[:] = jnp.concatenate([p[0:1], inter], axis=0)


def kernel(x, w, V, field_index):
    batch = x.shape[0]
    xt = x.T
    w2 = w.reshape(NUM_FEATURES, 1)
    fi2 = field_index.reshape(NUM_FEATURES, 1)
    grid = (batch // CB, EMBED // EH)
    yfm_t, out_p = pl.pallas_call(
        _fm_body,
        grid=grid,
        in_specs=[
            pl.BlockSpec(memory_space=pl.ANY),
            pl.BlockSpec((NUM_FEATURES, 1), lambda i, j: (0, 0)),
            pl.BlockSpec((NUM_FIELDS, EMBED), lambda i, j: (0, 0)),
            pl.BlockSpec((NUM_FEATURES, 1), lambda i, j: (0, 0)),
        ],
        out_specs=[
            pl.BlockSpec((2, CB), lambda i, j: (0, i)),
            pl.BlockSpec((NUM_FEATURES, EH, CB), lambda i, j: (0, j, i)),
        ],
        out_shape=[
            jax.ShapeDtypeStruct((2, batch), jnp.float32),
            jax.ShapeDtypeStruct((NUM_FEATURES, EMBED, batch), jnp.float32),
        ],
        scratch_shapes=[
            pltpu.VMEM((2, NUM_FEATURES, EH), jnp.float32),
            pltpu.VMEM((NUM_FEATURES, 2), jnp.float32),
            pltpu.VMEM((NUM_FEATURES, 1), jnp.float32),
            pltpu.VMEM((2, NUM_FEATURES, CB), jnp.float32),
            pltpu.SemaphoreType.DMA((2,)),
        ],
        compiler_params=pltpu.CompilerParams(
            dimension_semantics=("arbitrary", "arbitrary")),
    )(xt, w2, V, fi2)
    return (yfm_t.T, jnp.transpose(out_p, (2, 0, 1)))


# speedup vs baseline: 1.2397x; 1.2397x over previous
import jax
import jax.numpy as jnp
from jax import lax
from jax.experimental import pallas as pl
from jax.experimental.pallas import tpu as pltpu

NUM_FEATURES = 100
NUM_FIELDS = 26
EMBED = 16
CB = 1024


def _fm_body(xt_ref, w_ref, V_ref, fi_ref, yfm_ref, out_ref,
             emb_ref, a_ref, q_ref):
    f32 = jnp.float32
    hi = lax.Precision.HIGHEST

    @pl.when(pl.program_id(0) == 0)
    def _prep():
        fi = fi_ref[:]
        onehot = (fi == lax.broadcasted_iota(
            jnp.int32, (NUM_FEATURES, NUM_FIELDS), 1)).astype(f32)
        emb = jnp.dot(onehot, V_ref[:], precision=hi,
                      preferred_element_type=f32)
        emb_ref[:] = emb
        rowsum = jnp.sum(emb, axis=1, keepdims=True)
        a_ref[:] = jnp.concatenate([w_ref[:], rowsum], axis=1)
        q_ref[:] = jnp.sum(emb * emb, axis=1, keepdims=True)

    xtb = xt_ref[:]
    emb = emb_ref[:]
    for e in range(EMBED):
        out_ref[:, e, :] = xtb * emb[:, e:e + 1]

    p = lax.dot_general(a_ref[:], xtb, (((0,), (0,)), ((), ())),
                        precision=hi, preferred_element_type=f32)
    sq = lax.dot_general(q_ref[:], xtb * xtb, (((0,), (0,)), ((), ())),
                         precision=hi, preferred_element_type=f32)
    inter = 0.5 * (p[1:2] * p[1:2] - sq)
    yfm_ref[:] = jnp.concatenate([p[0:1], inter], axis=0)


def kernel(x, w, V, field_index):
    batch = x.shape[0]
    xt = x.T
    w2 = w.reshape(NUM_FEATURES, 1)
    fi2 = field_index.reshape(NUM_FEATURES, 1)
    grid = batch // CB
    yfm_t, out_p = pl.pallas_call(
        _fm_body,
        grid=(grid,),
        in_specs=[
            pl.BlockSpec((NUM_FEATURES, CB), lambda i: (0, i)),
            pl.BlockSpec((NUM_FEATURES, 1), lambda i: (0, 0)),
            pl.BlockSpec((NUM_FIELDS, EMBED), lambda i: (0, 0)),
            pl.BlockSpec((NUM_FEATURES, 1), lambda i: (0, 0)),
        ],
        out_specs=[
            pl.BlockSpec((2, CB), lambda i: (0, i)),
            pl.BlockSpec((NUM_FEATURES, EMBED, CB), lambda i: (0, 0, i)),
        ],
        out_shape=[
            jax.ShapeDtypeStruct((2, batch), jnp.float32),
            jax.ShapeDtypeStruct((NUM_FEATURES, EMBED, batch), jnp.float32),
        ],
        scratch_shapes=[
            pltpu.VMEM((NUM_FEATURES, EMBED), jnp.float32),
            pltpu.VMEM((NUM_FEATURES, 2), jnp.float32),
            pltpu.VMEM((NUM_FEATURES, 1), jnp.float32),
        ],
        compiler_params=pltpu.CompilerParams(
            dimension_semantics=("arbitrary",)),
    )(xt, w2, V, fi2)
    return (yfm_t.T, jnp.transpose(out_p, (2, 0, 1)))
